# Initial kernel scaffold; baseline (speedup 1.0000x reference)
import jax
import jax.numpy as jnp
from jax.experimental import pallas as pl


def kernel(target_x, target_edge_index, query_x, query_edge_index, mask, emb, Wl, bl, Wr, ntn_W, ntn_V, ntn_b, conv_w, conv_b):
    raise NotImplementedError("write your pallas kernel here")


# speedup vs baseline: 3.3133x; 3.3133x over previous
import math

import jax
import jax.numpy as jnp
from jax import lax
from jax.experimental import pallas as pl
from jax.experimental.pallas import tpu as pltpu
from jax.experimental.pallas import tpu_sc as plsc

_F32 = jnp.float32
_NC = 2
_NS = 16
_NW = _NC * _NS
_B = 128
_TB = 1024


def _dott(a, b):
    return lax.dot_general(a, b, (((1,), (1,)), ((), ())),
                           preferred_element_type=_F32)


def _dot(a, b):
    return jnp.dot(a, b, preferred_element_type=_F32)


def _act(h, layer, n_layers):
    if layer < n_layers - 1:
        return jnp.where(h > 0, h, jnp.exp(h) - 1.0)
    return 1.0 / (1.0 + jnp.exp(-h))


def _sc_segsum(x, src3, dst3, zrows, ntp, nch):
    nrows = ntp + _NS
    rps_z = nrows // _NS
    rps_o = ntp // _NS
    mesh = plsc.VectorSubcoreMesh(core_axis_name="c", subcore_axis_name="s")

    def body(x_hbm, src_hbm, dst_hbm, z_hbm, out_hbm,
             acc, srcv, dstv, gbuf, sg0, sg1):
        c = lax.axis_index("c")
        s = lax.axis_index("s")
        wid = s * _NC + c
        pltpu.sync_copy(z_hbm, acc.at[pl.ds(s * rps_z, rps_z)])
        pltpu.sync_copy(src_hbm.at[wid], srcv)
        pltpu.sync_copy(dst_hbm.at[wid], dstv)
        plsc.subcore_barrier()

        def step(i, carry):
            j0 = i * 2
            j1 = j0 + 1
            g0 = pltpu.async_copy(x_hbm.at[srcv.at[j0]], gbuf.at[0], sg0)
            g1 = pltpu.async_copy(x_hbm.at[srcv.at[j1]], gbuf.at[1], sg1)
            g0.wait()
            pltpu.sync_copy(gbuf.at[0], acc.at[dstv.at[j0]], add=True)
            g1.wait()
            pltpu.sync_copy(gbuf.at[1], acc.at[dstv.at[j1]], add=True)
            return carry

        lax.fori_loop(0, nch // 2, step, 0)
        plsc.subcore_barrier()
        pltpu.sync_copy(acc.at[pl.ds(s * rps_o, rps_o)],
                        out_hbm.at[c, pl.ds(s * rps_o, rps_o)])

    f = pl.kernel(
        body,
        out_type=jax.ShapeDtypeStruct((_NC, ntp, 128), _F32),
        mesh=mesh,
        scratch_types=[
            pltpu.VMEM_SHARED((nrows, 128), _F32),
            pltpu.VMEM((nch, _B), jnp.int32),
            pltpu.VMEM((nch, _B), jnp.int32),
            pltpu.VMEM((2, _B, 128), _F32),
            pltpu.SemaphoreType.DMA,
            pltpu.SemaphoreType.DMA,
        ],
    )
    return f(x, src3, dst3, zrows)


def _sc_deg(dst3, z16, ones16, ntp, nch):
    nrows = ntp + _NS
    rps_z = nrows // _NS
    rps_o = ntp // _NS
    mesh = plsc.VectorSubcoreMesh(core_axis_name="c", subcore_axis_name="s")

    def body(dst_hbm, z_hbm, ones_hbm, out_hbm, acc, dstv, ones_v):
        c = lax.axis_index("c")
        s = lax.axis_index("s")
        wid = s * _NC + c
        pltpu.sync_copy(z_hbm, acc.at[pl.ds(s * rps_z, rps_z)])
        pltpu.sync_copy(dst_hbm.at[wid], dstv)
        pltpu.sync_copy(ones_hbm, ones_v)
        plsc.subcore_barrier()

        def step(j, carry):
            pltpu.sync_copy(ones_v, acc.at[dstv.at[j]], add=True)
            return carry

        lax.fori_loop(0, nch, step, 0)
        plsc.subcore_barrier()
        pltpu.sync_copy(acc.at[pl.ds(s * rps_o, rps_o)],
                        out_hbm.at[c, pl.ds(s * rps_o, rps_o)])

    f = pl.kernel(
        body,
        out_type=jax.ShapeDtypeStruct((_NC, ntp, 16), _F32),
        mesh=mesh,
        scratch_types=[
            pltpu.VMEM_SHARED((nrows, 16), _F32),
            pltpu.VMEM((nch, _B), jnp.int32),
            pltpu.VMEM((_B, 16), _F32),
        ],
    )
    return f(dst3, z16, ones16)


def _tc_embed(tx_col, emb_pad, ntp):
    def body(tx_ref, emb_ref, o_ref):
        ids = tx_ref[...]
        io = lax.broadcasted_iota(jnp.int32, (_TB, 256), 1)
        oh = (ids == io).astype(_F32)
        o_ref[...] = _dot(oh, emb_ref[...])

    return pl.pallas_call(
        body,
        grid=(ntp // _TB,),
        in_specs=[pl.BlockSpec((_TB, 1), lambda t: (t, 0)),
                  pl.BlockSpec((256, 128), lambda t: (0, 0))],
        out_specs=pl.BlockSpec((_TB, 128), lambda t: (t, 0)),
        out_shape=jax.ShapeDtypeStruct((ntp, 128), _F32),
    )(tx_col, emb_pad)


def _tc_target_update(parts, degp, x, wl, blv, wr, layer, n_layers, ntp):
    def body(p_ref, d_ref, x_ref, wl_ref, bl_ref, wr_ref, o_ref):
        p = p_ref[...]
        ssum = p[0] + p[1]
        d = d_ref[...]
        deg = d[0, :, 0:1] + d[1, :, 0:1]
        mean = ssum / jnp.maximum(deg, 1.0)
        hh = _dott(mean, wl_ref[...]) + bl_ref[...] + _dott(x_ref[...], wr_ref[...])
        o_ref[...] = _act(hh, layer, n_layers)

    return pl.pallas_call(
        body,
        grid=(ntp // _TB,),
        in_specs=[
            pl.BlockSpec((2, _TB, 128), lambda t: (0, t, 0)),
            pl.BlockSpec((2, _TB, 16), lambda t: (0, t, 0)),
            pl.BlockSpec((_TB, 128), lambda t: (t, 0)),
            pl.BlockSpec((128, 128), lambda t: (0, 0)),
            pl.BlockSpec((1, 128), lambda t: (0, 0)),
            pl.BlockSpec((128, 128), lambda t: (0, 0)),
        ],
        out_specs=pl.BlockSpec((_TB, 128), lambda t: (t, 0)),
        out_shape=jax.ShapeDtypeStruct((ntp, 128), _F32),
    )(parts, degp, x, wl, blv, wr)


def _tc_query_update(eq, qs_col, qd_col, wl, blv, wr, emb_pad, qx_col,
                     layer, n_layers, eqn, nq):
    first = layer == 0

    def body(*refs):
        if first:
            qx_ref, emb_ref, s_ref, d_ref, wl_ref, bl_ref, wr_ref, o_ref = refs
            io = lax.broadcasted_iota(jnp.int32, (nq, 256), 1)
            x = _dot((qx_ref[...] == io).astype(_F32), emb_ref[...])
        else:
            x_ref, s_ref, d_ref, wl_ref, bl_ref, wr_ref, o_ref = refs
            x = x_ref[...]
        ios = lax.broadcasted_iota(jnp.int32, (eqn, nq), 1)
        ohs = (s_ref[...] == ios).astype(_F32)
        ohd = (d_ref[...] == ios).astype(_F32)
        gathered = _dot(ohs, x)
        ssum = lax.dot_general(ohd, gathered, (((0,), (0,)), ((), ())),
                               preferred_element_type=_F32)
        deg = lax.dot_general(ohd, jnp.ones((eqn, 8), _F32),
                              (((0,), (0,)), ((), ())),
                              preferred_element_type=_F32)[:, 0:1]
        hh = _dott(ssum / jnp.maximum(deg, 1.0), wl_ref[...]) + bl_ref[...] \
            + _dott(x, wr_ref[...])
        o_ref[...] = _act(hh, layer, n_layers)

    args = (qx_col, emb_pad, qs_col, qd_col, wl, blv, wr) if first \
        else (eq, qs_col, qd_col, wl, blv, wr)
    return pl.pallas_call(
        body,
        out_shape=jax.ShapeDtypeStruct((nq, 128), _F32),
    )(*args)


def _tc_att_stats(eq, et, mask_p, nt, ntp, nq):
    isq = 1.0 / math.sqrt(128.0)

    def body(eq_ref, et_ref, mk_ref, m_out, z_out, m_s, z_s):
        t = pl.program_id(0)

        @pl.when(t == 0)
        def _():
            m_s[...] = jnp.full((nq, 1), -1e30, _F32)
            z_s[...] = jnp.zeros((nq, 1), _F32)

        sc = _dott(eq_ref[...], et_ref[...])
        fm = mk_ref[...].astype(_F32)
        sc = sc * fm * isq + (-1e9) * (1.0 - fm)
        col = t * _TB + lax.broadcasted_iota(jnp.int32, sc.shape, 1)
        sc = jnp.where(col < nt, sc, -3e38)
        bm = jnp.max(sc, axis=1, keepdims=True)
        mold = m_s[...]
        mnew = jnp.maximum(mold, bm)
        z_s[...] = z_s[...] * jnp.exp(mold - mnew) \
            + jnp.sum(jnp.exp(sc - mnew), axis=1, keepdims=True)
        m_s[...] = mnew
        m_out[...] = mnew
        z_out[...] = z_s[...]

    return pl.pallas_call(
        body,
        grid=(ntp // _TB,),
        in_specs=[
            pl.BlockSpec((nq, 128), lambda t: (0, 0)),
            pl.BlockSpec((_TB, 128), lambda t: (t, 0)),
            pl.BlockSpec((nq, _TB), lambda t: (0, t)),
        ],
        out_specs=[pl.BlockSpec((nq, 1), lambda t: (0, 0)),
                   pl.BlockSpec((nq, 1), lambda t: (0, 0))],
        out_shape=[jax.ShapeDtypeStruct((nq, 1), _F32),
                   jax.ShapeDtypeStruct((nq, 1), _F32)],
        scratch_shapes=[pltpu.VMEM((nq, 1), _F32),
                        pltpu.VMEM((nq, 1), _F32)],
    )(eq, et, mask_p)


def _tc_att_acc(eq, et, mask_p, m, z, wn, vqw, vtw, nb, cw, acc,
                nt, ntp, nq, k_heads):
    isq = 1.0 / math.sqrt(128.0)

    def body(eq_ref, et_ref, mk_ref, m_ref, z_ref, wn_ref, vq_ref, vt_ref,
             nb_ref, cw_ref, ai_ref, ao_ref, t1_s):
        t = pl.program_id(0)
        eqv = eq_ref[...]

        @pl.when(t == 0)
        def _():
            for k in range(k_heads):
                t1_s[k] = _dot(eqv, wn_ref[k])

        etv = et_ref[...]
        sc = _dott(eqv, etv)
        fm = mk_ref[...].astype(_F32)
        sc = sc * fm * isq + (-1e9) * (1.0 - fm)
        col = t * _TB + lax.broadcasted_iota(jnp.int32, sc.shape, 1)
        sc = jnp.where(col < nt, sc, -3e38)
        att = jnp.exp(sc - m_ref[...]) / z_ref[...]
        vqa = _dott(eqv, vq_ref[...])
        vta = _dott(vt_ref[...], etv)
        nbv = nb_ref[...]
        cwv = cw_ref[...]
        contrib = None
        for k in range(k_heads):
            bil = _dott(t1_s[k], etv)
            ntn = jnp.maximum(bil + vqa[:, k:k + 1] + vta[k:k + 1, :]
                              + nbv[:, k:k + 1], 0.0)
            term = cwv[:, k:k + 1] * ntn
            contrib = term if contrib is None else contrib + term
        ao_ref[...] = ai_ref[...] + contrib * att

    return pl.pallas_call(
        body,
        grid=(ntp // _TB,),
        in_specs=[
            pl.BlockSpec((nq, 128), lambda t: (0, 0)),
            pl.BlockSpec((_TB, 128), lambda t: (t, 0)),
            pl.BlockSpec((nq, _TB), lambda t: (0, t)),
            pl.BlockSpec((nq, 1), lambda t: (0, 0)),
            pl.BlockSpec((nq, 1), lambda t: (0, 0)),
            pl.BlockSpec((k_heads, 128, 128), lambda t: (0, 0, 0)),
            pl.BlockSpec((k_heads, 128), lambda t: (0, 0)),
            pl.BlockSpec((k_heads, 128), lambda t: (0, 0)),
            pl.BlockSpec((1, k_heads), lambda t: (0, 0)),
            pl.BlockSpec((1, k_heads), lambda t: (0, 0)),
            pl.BlockSpec((nq, _TB), lambda t: (0, t)),
        ],
        out_specs=pl.BlockSpec((nq, _TB), lambda t: (0, t)),
        out_shape=jax.ShapeDtypeStruct((nq, ntp), _F32),
        scratch_shapes=[pltpu.VMEM((k_heads, nq, 128), _F32)],
        input_output_aliases={10: 0},
    )(eq, et, mask_p, m, z, wn, vqw, vtw, nb, cw, acc)


def _tc_final_stats(acc, nt, ntp, nq):
    def body(a_ref, m_out, z_out, m_s, z_s):
        t = pl.program_id(0)

        @pl.when(t == 0)
        def _():
            m_s[...] = jnp.full((nq, 1), -1e30, _F32)
            z_s[...] = jnp.zeros((nq, 1), _F32)

        sc = a_ref[...]
        col = t * _TB + lax.broadcasted_iota(jnp.int32, sc.shape, 1)
        sc = jnp.where(col < nt, sc, -3e38)
        bm = jnp.max(sc, axis=1, keepdims=True)
        mold = m_s[...]
        mnew = jnp.maximum(mold, bm)
        z_s[...] = z_s[...] * jnp.exp(mold - mnew) \
            + jnp.sum(jnp.exp(sc - mnew), axis=1, keepdims=True)
        m_s[...] = mnew
        m_out[...] = mnew
        z_out[...] = z_s[...]

    return pl.pallas_call(
        body,
        grid=(ntp // _TB,),
        in_specs=[pl.BlockSpec((nq, _TB), lambda t: (0, t))],
        out_specs=[pl.BlockSpec((nq, 1), lambda t: (0, 0)),
                   pl.BlockSpec((nq, 1), lambda t: (0, 0))],
        out_shape=[jax.ShapeDtypeStruct((nq, 1), _F32),
                   jax.ShapeDtypeStruct((nq, 1), _F32)],
        scratch_shapes=[pltpu.VMEM((nq, 1), _F32),
                        pltpu.VMEM((nq, 1), _F32)],
    )(acc)


def _tc_final_out(acc, m, z, ntp, nq):
    def body(a_ref, m_ref, z_ref, o_ref):
        o_ref[...] = jnp.exp(a_ref[...] - m_ref[...]) / z_ref[...]

    return pl.pallas_call(
        body,
        grid=(ntp // _TB,),
        in_specs=[
            pl.BlockSpec((nq, _TB), lambda t: (0, t)),
            pl.BlockSpec((nq, 1), lambda t: (0, 0)),
            pl.BlockSpec((nq, 1), lambda t: (0, 0)),
        ],
        out_specs=pl.BlockSpec((nq, _TB), lambda t: (0, t)),
        out_shape=jax.ShapeDtypeStruct((nq, ntp), _F32),
    )(acc, m, z)


def kernel(target_x, target_edge_index, query_x, query_edge_index, mask,
           emb, Wl, bl, Wr, ntn_W, ntn_V, ntn_b, conv_w, conv_b):
    nt = target_x.shape[0]
    nq = query_x.shape[0]
    et_n = target_edge_index.shape[1]
    eq_n = query_edge_index.shape[1]
    hdim = emb.shape[1]
    n_layers = Wl.shape[0]
    k_heads = ntn_W.shape[1]

    ntp = -(-nt // _TB) * _TB
    emb_pad = jnp.zeros((256, hdim), _F32).at[: emb.shape[0]].set(
        emb.astype(_F32))
    tx_col = jnp.pad(target_x.astype(jnp.int32).reshape(nt, 1),
                     ((0, ntp - nt), (0, 0)))
    qx_col = query_x.astype(jnp.int32).reshape(nq, 1)

    nch = -(-et_n // (_NW * _B))
    nch += nch % 2
    tot = _NW * nch * _B
    src = target_edge_index[0].astype(jnp.int32)
    dst = target_edge_index[1].astype(jnp.int32)
    src3 = jnp.concatenate(
        [src, jnp.zeros((tot - et_n,), jnp.int32)]).reshape(_NW, nch, _B)
    dst3 = jnp.concatenate(
        [dst, jnp.full((tot - et_n,), ntp, jnp.int32)]).reshape(_NW, nch, _B)

    nrows = ntp + _NS
    z128 = jnp.zeros((nrows // _NS, hdim), _F32)
    z16 = jnp.zeros((nrows // _NS, 16), _F32)
    ones16 = jnp.ones((_B, 16), _F32)

    degp = _sc_deg(dst3, z16, ones16, ntp, nch)

    qs_col = query_edge_index[0].astype(jnp.int32).reshape(eq_n, 1)
    qd_col = query_edge_index[1].astype(jnp.int32).reshape(eq_n, 1)
    mask_p = jnp.pad(mask, ((0, 0), (0, ntp - nt)))

    vq_w = ntn_V[:, :, :hdim]
    vt_w = ntn_V[:, :, hdim:]

    x_t = _tc_embed(tx_col, emb_pad, ntp)
    x_q = None
    acc = jnp.zeros((nq, ntp), _F32)
    for l in range(n_layers):
        parts = _sc_segsum(x_t, src3, dst3, z128, ntp, nch)
        x_t_new = _tc_target_update(parts, degp, x_t, Wl[l],
                                    bl[l].reshape(1, hdim), Wr[l],
                                    l, n_layers, ntp)
        x_q = _tc_query_update(x_q, qs_col, qd_col, Wl[l],
                               bl[l].reshape(1, hdim), Wr[l],
                               emb_pad, qx_col, l, n_layers, eq_n, nq)
        m, z = _tc_att_stats(x_q, x_t_new, mask_p, nt, ntp, nq)
        acc = _tc_att_acc(x_q, x_t_new, mask_p, m, z, ntn_W[l],
                          vq_w[l], vt_w[l], ntn_b[l].reshape(1, k_heads),
                          lax.dynamic_slice(conv_w, (l * k_heads,),
                                            (k_heads,)).reshape(1, k_heads),
                          acc, nt, ntp, nq, k_heads)
        x_t = x_t_new
    m2, z2 = _tc_final_stats(acc, nt, ntp, nq)
    out = _tc_final_out(acc, m2, z2, ntp, nq)
    return out[:, :nt][None]
